# baseline (speedup 1.0000x reference)
import functools

import jax
import jax.numpy as jnp
from jax import lax
from jax.experimental import pallas as pl
from jax.experimental.pallas import tpu as pltpu
from jax.experimental.pallas import tpu_sc as plsc

TOPK = 64
NEG = -1e30


def _router_kernel(x_ref, wr_ref, out_ref):
    xb = x_ref[0]
    wr = wr_ref[...]
    s = lax.dot_general(wr, xb, (((1,), (1,)), ((), ())),
                        preferred_element_type=jnp.float32)
    out_ref[0, :, 0, :] = s


def _router_scores(x, Wr, TB=512):
    B, T, D = x.shape
    H = Wr.shape[0]
    return pl.pallas_call(
        _router_kernel,
        grid=(B, T // TB),
        in_specs=[
            pl.BlockSpec((1, TB, D), lambda b, t: (b, t, 0)),
            pl.BlockSpec((H, D), lambda b, t: (0, 0)),
        ],
        out_specs=pl.BlockSpec((1, H, 1, TB), lambda b, t: (b, 0, 0, t)),
        out_shape=jax.ShapeDtypeStruct((B, H, 1, T), jnp.float32),
    )(x, Wr)


def _topk_kernel(s_ref, out_ref, *, T, K):
    R = T // 128
    a = s_ref[0, 0, 0, :].reshape(R, 128)
    pos = lax.broadcasted_iota(jnp.int32, (R, 128), 0) * 128 + \
        lax.broadcasted_iota(jnp.int32, (R, 128), 1)
    kiota = lax.broadcasted_iota(jnp.int32, (1, K), 1)
    acc = jnp.zeros((1, K), jnp.int32)
    for k in range(K):
        m = jnp.max(a)
        cand = jnp.where(a == m, pos, T)
        i = jnp.min(cand)
        acc = jnp.where(kiota == k, i, acc)
        a = jnp.where(pos == i, NEG, a)
    out_ref[0, 0] = acc


def _topk(scores, K):
    B, H, _, T = scores.shape
    return pl.pallas_call(
        functools.partial(_topk_kernel, T=T, K=K),
        grid=(B, H),
        in_specs=[pl.BlockSpec((1, 1, 1, T), lambda b, h: (b, h, 0, 0))],
        out_specs=pl.BlockSpec((1, 1, 1, K), lambda b, h: (b, h, 0, 0)),
        out_shape=jax.ShapeDtypeStruct((B, H, 1, K), jnp.int32),
    )(scores)


def _sc_gather(x2, idx):
    R, = idx.shape
    D = x2.shape[1]
    info = plsc.get_sparse_core_info()
    NC, NS = info.num_cores, info.num_subcores
    NW = NC * NS
    rpw = R // NW
    mesh = plsc.VectorSubcoreMesh(core_axis_name="c", subcore_axis_name="s")

    @functools.partial(
        pl.kernel, mesh=mesh,
        out_type=jax.ShapeDtypeStruct((R, D), jnp.float32),
        scratch_types=[
            pltpu.VMEM((rpw,), jnp.int32),
            pltpu.VMEM((rpw, D), jnp.float32),
            pltpu.SemaphoreType.DMA,
        ],
    )
    def kg(x_hbm, idx_hbm, out_hbm, idx_v, rows_v, sem):
        wid = lax.axis_index("s") * NC + lax.axis_index("c")
        base = wid * rpw
        pltpu.sync_copy(idx_hbm.at[pl.ds(base, rpw)], idx_v)
        pltpu.async_copy(x_hbm.at[idx_v], rows_v, sem).wait()
        pltpu.sync_copy(rows_v, out_hbm.at[pl.ds(base, rpw)])

    return kg(x2, idx)


def _attn_kernel(xs_ref, wq_ref, wk_ref, wv_ref, wo_ref, ir_ref, ic_ref,
                 out_ref, *, scale):
    xs = xs_ref[0]
    cdn = (((1,), (1,)), ((), ()))
    q = lax.dot_general(xs, wq_ref[0], cdn,
                        preferred_element_type=jnp.float32)
    k = lax.dot_general(xs, wk_ref[0], cdn,
                        preferred_element_type=jnp.float32)
    v = lax.dot_general(xs, wv_ref[0], cdn,
                        preferred_element_type=jnp.float32)
    s = lax.dot_general(q, k, cdn,
                        preferred_element_type=jnp.float32) * scale
    trow = ir_ref[0]
    tcol = ic_ref[0]
    s = jnp.where(tcol >= trow, s, NEG)
    m = jnp.max(s, axis=-1, keepdims=True)
    e = jnp.exp(s - m)
    p = e / jnp.sum(e, axis=-1, keepdims=True)
    o = jnp.dot(p, v, preferred_element_type=jnp.float32)
    out_ref[0] = jnp.dot(o, wo_ref[0],
                         preferred_element_type=jnp.float32)


def _head_attention(xsel, Wq3, Wk3, Wv3, WoT3, idx_row, idx_col, K, DH):
    BH, _, D = xsel.shape
    H = Wq3.shape[0]
    wspec = pl.BlockSpec((1, DH, D), lambda bh: (bh % H, 0, 0))
    return pl.pallas_call(
        functools.partial(_attn_kernel, scale=DH ** -0.5),
        grid=(BH,),
        in_specs=[
            pl.BlockSpec((1, K, D), lambda bh: (bh, 0, 0)),
            wspec, wspec, wspec, wspec,
            pl.BlockSpec((1, 1, K), lambda bh: (bh, 0, 0)),
            pl.BlockSpec((1, K, 1), lambda bh: (bh, 0, 0)),
        ],
        out_specs=pl.BlockSpec((1, K, D), lambda bh: (bh, 0, 0)),
        out_shape=jax.ShapeDtypeStruct((BH, K, D), jnp.float32),
    )(xsel, Wq3, Wk3, Wv3, WoT3, idx_row, idx_col)


def _scatter_kernel(idx_sref, c_ref, out_ref, *, H, K, CH):
    b = pl.program_id(0)
    c = pl.program_id(1)
    h = pl.program_id(2)

    @pl.when(h == 0)
    def _():
        out_ref[...] = jnp.zeros_like(out_ref)

    def body(kk, _):
        t = idx_sref[(b * H + h) * K + kk]
        loc = t - c * CH

        @pl.when((loc >= 0) & (loc < CH))
        def _():
            out_ref[0, pl.ds(loc, 1), :] = (
                out_ref[0, pl.ds(loc, 1), :] + c_ref[0, pl.ds(kk, 1), :])
        return 0

    lax.fori_loop(0, K, body, 0)


def _scatter(contrib, idx_flat, B, T, D, H, K, CH=2048):
    grid_spec = pltpu.PrefetchScalarGridSpec(
        num_scalar_prefetch=1,
        grid=(B, T // CH, H),
        in_specs=[pl.BlockSpec((1, K, D), lambda b, c, h, i: (b * H + h, 0, 0))],
        out_specs=pl.BlockSpec((1, CH, D), lambda b, c, h, i: (b, c, 0)),
    )
    return pl.pallas_call(
        functools.partial(_scatter_kernel, H=H, K=K, CH=CH),
        grid_spec=grid_spec,
        out_shape=jax.ShapeDtypeStruct((B, T, D), jnp.float32),
    )(idx_flat, contrib)


def kernel(x, Wq, Wk, Wv, Wo, Wr):
    B, T, D = x.shape
    H = Wr.shape[0]
    DH = D // H
    K = min(TOPK, T)

    scores = _router_scores(x, Wr)
    idx = _topk(scores, K)
    idx3 = idx.reshape(B, H, K)

    offs = jnp.arange(B, dtype=jnp.int32)[:, None, None] * T
    idx_flat = (idx3 + offs).reshape(B * H * K)
    xsel = _sc_gather(x.reshape(B * T, D), idx_flat)

    idx_f = idx3.astype(jnp.float32)
    contrib = _head_attention(
        xsel.reshape(B * H, K, D),
        Wq.reshape(H, DH, D), Wk.reshape(H, DH, D), Wv.reshape(H, DH, D),
        Wo.T.reshape(H, DH, D),
        idx_f.reshape(B * H, 1, K),
        idx_f.reshape(B * H, K, 1),
        K, DH)

    return _scatter(contrib, idx3.reshape(B * H * K), B, T, D, H, K)
